# Initial kernel scaffold; baseline (speedup 1.0000x reference)
import jax
import jax.numpy as jnp
from jax.experimental import pallas as pl


def kernel(embeddings, edge_index, W, r_id):
    raise NotImplementedError("write your pallas kernel here")


# speedup vs baseline: 1.6267x; 1.6267x over previous
import dataclasses
import functools

import jax
import jax.numpy as jnp
from jax import lax
from jax.experimental import pallas as pl
from jax.experimental.pallas import tpu as pltpu
from jax.experimental.pallas import tpu_sc as plsc

DIM = 64
BLK = 128
LANES = 16
NUM_WORKERS = 32
MM_BLK = 1000


def _mm_kernel(x_ref, w_ref, o_ref):
    o_ref[...] = jnp.dot(x_ref[...], w_ref[...],
                         preferred_element_type=jnp.float32)


def _transform(emb, w_r):
    n, d = emb.shape
    return pl.pallas_call(
        _mm_kernel,
        grid=(n // MM_BLK,),
        in_specs=[
            pl.BlockSpec((MM_BLK, d), lambda i: (i, 0)),
            pl.BlockSpec((d, d), lambda i: (0, 0)),
        ],
        out_specs=pl.BlockSpec((MM_BLK, d), lambda i: (i, 0)),
        out_shape=jax.ShapeDtypeStruct((n, d), jnp.float32),
    )(emb, w_r)


def _sc_scores(tab_left, tab_right, left2d, right2d):
    nb = left2d.shape[0]
    mesh = plsc.VectorSubcoreMesh(core_axis_name="c", subcore_axis_name="s")
    cp = pltpu.CompilerParams()
    for fld, val in (("needs_layout_passes", False),
                     ("use_tc_tiling_on_sc", False)):
        if fld in pltpu.CompilerParams.__dataclass_fields__:
            cp = dataclasses.replace(cp, **{fld: val})

    @functools.partial(
        pl.kernel,
        mesh=mesh,
        compiler_params=cp,
        out_type=jax.ShapeDtypeStruct((nb, BLK), jnp.float32),
        scratch_types=[
            pltpu.VMEM((BLK,), jnp.int32),
            pltpu.VMEM((BLK,), jnp.int32),
            pltpu.VMEM((BLK, DIM), jnp.float32),
            pltpu.VMEM((BLK, DIM), jnp.float32),
            pltpu.VMEM((BLK,), jnp.float32),
            pltpu.SemaphoreType.DMA,
            pltpu.SemaphoreType.DMA,
        ],
    )
    def k(tl_hbm, tr_hbm, li_hbm, ri_hbm, o_hbm,
          li_v, ri_v, ra_v, rb_v, o_v, sem_a, sem_b):
        wid = lax.axis_index("s") * 2 + lax.axis_index("c")
        lo = (wid * nb) // NUM_WORKERS
        hi = ((wid + 1) * nb) // NUM_WORKERS

        @pl.loop(lo, hi)
        def _(blk):
            pltpu.sync_copy(li_hbm.at[blk], li_v)
            pltpu.sync_copy(ri_hbm.at[blk], ri_v)
            ca = pltpu.async_copy(tl_hbm.at[li_v], ra_v, sem_a)
            cb = pltpu.async_copy(tr_hbm.at[ri_v], rb_v, sem_b)
            ca.wait()
            cb.wait()

            @pl.loop(0, BLK, step=LANES)
            def _(base):
                rows = base + lax.broadcasted_iota(jnp.int32, (LANES,), 0)
                acc = jnp.zeros((LANES,), jnp.float32)
                for dcol in range(DIM):
                    cols = jnp.full((LANES,), dcol, jnp.int32)
                    a = plsc.load_gather(ra_v, [rows, cols])
                    b = plsc.load_gather(rb_v, [rows, cols])
                    acc = acc + a * b
                o_v[pl.ds(base, LANES)] = acc

            pltpu.sync_copy(o_v, o_hbm.at[blk])

    return k(tab_left, tab_right, left2d, right2d)


def kernel(embeddings, edge_index, W, r_id):
    n, d = embeddings.shape
    num_e = edge_index.shape[1]
    w_r = jnp.asarray(W)[r_id]
    tab_left = _transform(embeddings, w_r)
    nb = num_e // BLK
    left = edge_index[0].reshape(nb, BLK)
    right = edge_index[1].reshape(nb, BLK)
    out = _sc_scores(tab_left, embeddings, left, right)
    return out.reshape(num_e)


# speedup vs baseline: 1.9863x; 1.2211x over previous
import dataclasses
import functools

import jax
import jax.numpy as jnp
from jax import lax
from jax.experimental import pallas as pl
from jax.experimental.pallas import tpu as pltpu
from jax.experimental.pallas import tpu_sc as plsc

DIM = 64
BLK = 128
LANES = 16
NUM_WORKERS = 32
CHUNK = 28
BPW = 196
MM_BLK = 1000


def _mm_kernel(x_ref, w_ref, o_ref):
    o_ref[...] = jnp.dot(x_ref[...], w_ref[...],
                         preferred_element_type=jnp.float32)


def _transform(emb, w_r):
    n, d = emb.shape
    return pl.pallas_call(
        _mm_kernel,
        grid=(n // MM_BLK,),
        in_specs=[
            pl.BlockSpec((MM_BLK, d), lambda i: (i, 0)),
            pl.BlockSpec((d, d), lambda i: (0, 0)),
        ],
        out_specs=pl.BlockSpec((MM_BLK, d), lambda i: (i, 0)),
        out_shape=jax.ShapeDtypeStruct((n, d), jnp.float32),
    )(emb, w_r)


def _block_dot(ra_v, rb_v, o_v, orow):
    @pl.loop(0, BLK, step=LANES)
    def _(base):
        rows = base + lax.broadcasted_iota(jnp.int32, (LANES,), 0)
        acc = jnp.zeros((LANES,), jnp.float32)
        for dcol in range(DIM):
            cols = jnp.full((LANES,), dcol, jnp.int32)
            a = plsc.load_gather(ra_v, [rows, cols])
            b = plsc.load_gather(rb_v, [rows, cols])
            acc = acc + a * b
        o_v[orow, pl.ds(base, LANES)] = acc


def _sc_scores(tab_left, tab_right, left2d, right2d):
    nb = left2d.shape[0]
    mesh = plsc.VectorSubcoreMesh(core_axis_name="c", subcore_axis_name="s")
    cp = pltpu.CompilerParams()
    for fld, val in (("needs_layout_passes", False),
                     ("use_tc_tiling_on_sc", False)):
        if fld in pltpu.CompilerParams.__dataclass_fields__:
            cp = dataclasses.replace(cp, **{fld: val})

    @functools.partial(
        pl.kernel,
        mesh=mesh,
        compiler_params=cp,
        out_type=jax.ShapeDtypeStruct((nb, BLK), jnp.float32),
        scratch_types=[
            pltpu.VMEM((CHUNK, BLK), jnp.int32),
            pltpu.VMEM((CHUNK, BLK), jnp.int32),
            pltpu.VMEM((BLK, DIM), jnp.float32),
            pltpu.VMEM((BLK, DIM), jnp.float32),
            pltpu.VMEM((BLK, DIM), jnp.float32),
            pltpu.VMEM((BLK, DIM), jnp.float32),
            pltpu.VMEM((CHUNK, BLK), jnp.float32),
            pltpu.SemaphoreType.DMA,
            pltpu.SemaphoreType.DMA,
            pltpu.SemaphoreType.DMA,
            pltpu.SemaphoreType.DMA,
        ],
    )
    def k(tl_hbm, tr_hbm, li_hbm, ri_hbm, o_hbm,
          li_v, ri_v, ra0_v, ra1_v, rb0_v, rb1_v, o_v,
          sa0, sa1, sb0, sb1):
        wid = lax.axis_index("s") * 2 + lax.axis_index("c")
        lo = wid * BPW

        @pl.loop(0, BPW // CHUNK)
        def _(c):
            start = lo + c * CHUNK
            pltpu.sync_copy(li_hbm.at[pl.ds(start, CHUNK)], li_v)
            pltpu.sync_copy(ri_hbm.at[pl.ds(start, CHUNK)], ri_v)

            pltpu.async_copy(tl_hbm.at[li_v.at[0]], ra0_v, sa0)
            pltpu.async_copy(tr_hbm.at[ri_v.at[0]], rb0_v, sb0)

            @pl.loop(0, CHUNK, step=2)
            def _(j):
                pltpu.async_copy(tl_hbm.at[li_v.at[j + 1]], ra1_v, sa1)
                pltpu.async_copy(tr_hbm.at[ri_v.at[j + 1]], rb1_v, sb1)
                pltpu.make_async_copy(tl_hbm.at[li_v.at[j]], ra0_v, sa0).wait()
                pltpu.make_async_copy(tr_hbm.at[ri_v.at[j]], rb0_v, sb0).wait()
                _block_dot(ra0_v, rb0_v, o_v, j)

                @pl.when(j + 2 < CHUNK)
                def _():
                    pltpu.async_copy(tl_hbm.at[li_v.at[j + 2]], ra0_v, sa0)
                    pltpu.async_copy(tr_hbm.at[ri_v.at[j + 2]], rb0_v, sb0)

                pltpu.make_async_copy(
                    tl_hbm.at[li_v.at[j + 1]], ra1_v, sa1).wait()
                pltpu.make_async_copy(
                    tr_hbm.at[ri_v.at[j + 1]], rb1_v, sb1).wait()
                _block_dot(ra1_v, rb1_v, o_v, j + 1)

            pltpu.sync_copy(o_v, o_hbm.at[pl.ds(start, CHUNK)])

    return k(tab_left, tab_right, left2d, right2d)


def kernel(embeddings, edge_index, W, r_id):
    n, d = embeddings.shape
    num_e = edge_index.shape[1]
    w_r = jnp.asarray(W)[r_id]
    tab_left = _transform(embeddings, w_r)
    nb = num_e // BLK
    nb_pad = NUM_WORKERS * BPW
    left = edge_index[0].reshape(nb, BLK)
    right = edge_index[1].reshape(nb, BLK)
    pad = ((0, nb_pad - nb), (0, 0))
    left = jnp.pad(left, pad)
    right = jnp.pad(right, pad)
    out = _sc_scores(tab_left, embeddings, left, right)
    return out[:nb].reshape(num_e)


# speedup vs baseline: 5.8568x; 2.9486x over previous
import dataclasses
import functools

import jax
import jax.numpy as jnp
from jax import lax
from jax.experimental import pallas as pl
from jax.experimental.pallas import tpu as pltpu
from jax.experimental.pallas import tpu_sc as plsc

DIM = 64
BLK = 128
LANES = 16
NUM_WORKERS = 32
CHUNK = 28
BPW = 196
MM_BLK = 1000


def _mm_kernel(x_ref, w_ref, o_ref):
    o_ref[...] = jnp.dot(x_ref[...], w_ref[...],
                         preferred_element_type=jnp.float32)


def _transform(emb, w_r):
    n, d = emb.shape
    return pl.pallas_call(
        _mm_kernel,
        grid=(n // MM_BLK,),
        in_specs=[
            pl.BlockSpec((MM_BLK, d), lambda i: (i, 0)),
            pl.BlockSpec((d, d), lambda i: (0, 0)),
        ],
        out_specs=pl.BlockSpec((MM_BLK, d), lambda i: (i, 0)),
        out_shape=jax.ShapeDtypeStruct((n, d), jnp.float32),
    )(emb, w_r)


def _block_dot(ra_v, rb_v, o_v, orow):
    @pl.loop(0, BLK, step=LANES)
    def _(base):
        lane = lax.broadcasted_iota(jnp.int32, (LANES,), 0)
        rows = base + lane
        accs = [jnp.zeros((LANES,), jnp.float32) for _ in range(4)]
        for dcol in range(DIM):
            cols = (lane + dcol) & (DIM - 1)
            a = plsc.load_gather(ra_v, [rows, cols])
            b = plsc.load_gather(rb_v, [rows, cols])
            accs[dcol % 4] = accs[dcol % 4] + a * b
        acc = (accs[0] + accs[1]) + (accs[2] + accs[3])
        o_v[orow, pl.ds(base, LANES)] = acc


def _sc_scores(tab_left, tab_right, left2d, right2d):
    nb = left2d.shape[0]
    mesh = plsc.VectorSubcoreMesh(core_axis_name="c", subcore_axis_name="s")
    cp = pltpu.CompilerParams()
    for fld, val in (("needs_layout_passes", False),
                     ("use_tc_tiling_on_sc", False)):
        if fld in pltpu.CompilerParams.__dataclass_fields__:
            cp = dataclasses.replace(cp, **{fld: val})

    @functools.partial(
        pl.kernel,
        mesh=mesh,
        compiler_params=cp,
        out_type=jax.ShapeDtypeStruct((nb, BLK), jnp.float32),
        scratch_types=[
            pltpu.VMEM((CHUNK, BLK), jnp.int32),
            pltpu.VMEM((CHUNK, BLK), jnp.int32),
            pltpu.VMEM((BLK, DIM), jnp.float32),
            pltpu.VMEM((BLK, DIM), jnp.float32),
            pltpu.VMEM((BLK, DIM), jnp.float32),
            pltpu.VMEM((BLK, DIM), jnp.float32),
            pltpu.VMEM((CHUNK, BLK), jnp.float32),
            pltpu.SemaphoreType.DMA,
            pltpu.SemaphoreType.DMA,
            pltpu.SemaphoreType.DMA,
            pltpu.SemaphoreType.DMA,
        ],
    )
    def k(tl_hbm, tr_hbm, li_hbm, ri_hbm, o_hbm,
          li_v, ri_v, ra0_v, ra1_v, rb0_v, rb1_v, o_v,
          sa0, sa1, sb0, sb1):
        wid = lax.axis_index("s") * 2 + lax.axis_index("c")
        lo = wid * BPW

        @pl.loop(0, BPW // CHUNK)
        def _(c):
            start = lo + c * CHUNK
            pltpu.sync_copy(li_hbm.at[pl.ds(start, CHUNK)], li_v)
            pltpu.sync_copy(ri_hbm.at[pl.ds(start, CHUNK)], ri_v)

            pltpu.async_copy(tl_hbm.at[li_v.at[0]], ra0_v, sa0)
            pltpu.async_copy(tr_hbm.at[ri_v.at[0]], rb0_v, sb0)

            @pl.loop(0, CHUNK, step=2)
            def _(j):
                pltpu.async_copy(tl_hbm.at[li_v.at[j + 1]], ra1_v, sa1)
                pltpu.async_copy(tr_hbm.at[ri_v.at[j + 1]], rb1_v, sb1)
                pltpu.make_async_copy(tl_hbm.at[li_v.at[j]], ra0_v, sa0).wait()
                pltpu.make_async_copy(tr_hbm.at[ri_v.at[j]], rb0_v, sb0).wait()
                _block_dot(ra0_v, rb0_v, o_v, j)

                @pl.when(j + 2 < CHUNK)
                def _():
                    pltpu.async_copy(tl_hbm.at[li_v.at[j + 2]], ra0_v, sa0)
                    pltpu.async_copy(tr_hbm.at[ri_v.at[j + 2]], rb0_v, sb0)

                pltpu.make_async_copy(
                    tl_hbm.at[li_v.at[j + 1]], ra1_v, sa1).wait()
                pltpu.make_async_copy(
                    tr_hbm.at[ri_v.at[j + 1]], rb1_v, sb1).wait()
                _block_dot(ra1_v, rb1_v, o_v, j + 1)

            pltpu.sync_copy(o_v, o_hbm.at[pl.ds(start, CHUNK)])

    return k(tab_left, tab_right, left2d, right2d)


def kernel(embeddings, edge_index, W, r_id):
    n, d = embeddings.shape
    num_e = edge_index.shape[1]
    w_r = jnp.asarray(W)[r_id]
    tab_left = _transform(embeddings, w_r)
    nb = num_e // BLK
    nb_pad = NUM_WORKERS * BPW
    left = edge_index[0].reshape(nb, BLK)
    right = edge_index[1].reshape(nb, BLK)
    pad = ((0, nb_pad - nb), (0, 0))
    left = jnp.pad(left, pad)
    right = jnp.pad(right, pad)
    out = _sc_scores(tab_left, embeddings, left, right)
    return out[:nb].reshape(num_e)
